# Initial kernel scaffold; baseline (speedup 1.0000x reference)
import functools

import jax
import jax.numpy as jnp
from jax import lax
from jax.experimental import pallas as pl
from jax.experimental.pallas import tpu as pltpu
from jax.experimental.pallas import tpu_sc as plsc

_NC = 2
_NS = 16
_NW = _NC * _NS


def _node_proj(x, Wab):
    n, nd = x.shape
    h2 = Wab.shape[1]
    blk = 1000

    def body(x_ref, w_ref, xs_ref, xd_ref):
        xw = jnp.dot(x_ref[...], w_ref[...], preferred_element_type=jnp.float32)
        xs_ref[...] = xw[:, : h2 // 2]
        xd_ref[...] = xw[:, h2 // 2 :]

    return pl.pallas_call(
        body,
        grid=(n // blk,),
        in_specs=[
            pl.BlockSpec((blk, nd), lambda i: (i, 0)),
            pl.BlockSpec((nd, h2), lambda i: (0, 0)),
        ],
        out_specs=[
            pl.BlockSpec((blk, h2 // 2), lambda i: (i, 0)),
            pl.BlockSpec((blk, h2 // 2), lambda i: (i, 0)),
        ],
        out_shape=[
            jax.ShapeDtypeStruct((n, h2 // 2), jnp.float32),
            jax.ShapeDtypeStruct((n, h2 // 2), jnp.float32),
        ],
    )(x, Wab)


def _sc_gather(xs, xd, row, col):
    e = row.shape[0]
    hid = xs.shape[1]
    epw = e // _NW
    ch = 80
    nch = epw // ch
    assert epw % ch == 0

    mesh = plsc.VectorSubcoreMesh(core_axis_name="c", subcore_axis_name="s")

    @functools.partial(
        pl.kernel,
        mesh=mesh,
        out_type=jax.ShapeDtypeStruct((e, hid), jnp.float32),
        scratch_types=[
            pltpu.VMEM((ch,), jnp.int32),
            pltpu.VMEM((ch,), jnp.int32),
            pltpu.VMEM((ch, hid), jnp.float32),
        ],
    )
    def k(xs_hbm, xd_hbm, row_hbm, col_hbm, g_hbm, rowi, coli, gbuf):
        wid = lax.axis_index("s") * _NC + lax.axis_index("c")

        def body(i, carry):
            base = wid * epw + i * ch
            pltpu.sync_copy(row_hbm.at[pl.ds(base, ch)], rowi)
            pltpu.sync_copy(col_hbm.at[pl.ds(base, ch)], coli)
            pltpu.sync_copy(xs_hbm.at[rowi], gbuf)
            pltpu.sync_copy(xd_hbm.at[coli], gbuf, add=True)
            pltpu.sync_copy(gbuf, g_hbm.at[pl.ds(base, ch)])
            return carry

        lax.fori_loop(0, nch, body, 0)

    return k(xs, xd, row, col)


def _edge_mlp(g, ea, W1c, b1, W2, b2, ge2, be2ln):
    e, hid = g.shape
    ed = ea.shape[1]
    eb = 3200

    def body(g_ref, ea_ref, w1_ref, b1_ref, w2_ref, b2_ref, ge_ref, be_ref,
             e2_ref, ln_ref):
        eav = ea_ref[...]
        h1 = g_ref[...] + jnp.dot(eav, w1_ref[...],
                                  preferred_element_type=jnp.float32) + b1_ref[...]
        h = jnp.dot(jnp.maximum(h1, 0.0), w2_ref[...],
                    preferred_element_type=jnp.float32) + b2_ref[...] + eav
        e2_ref[...] = h
        m = jnp.mean(h, axis=-1, keepdims=True)
        v = jnp.mean((h - m) ** 2, axis=-1, keepdims=True)
        ln_ref[...] = (h - m) * lax.rsqrt(v + 1e-5) * ge_ref[...] + be_ref[...]

    return pl.pallas_call(
        body,
        grid=(e // eb,),
        in_specs=[
            pl.BlockSpec((eb, hid), lambda i: (i, 0)),
            pl.BlockSpec((eb, ed), lambda i: (i, 0)),
            pl.BlockSpec((ed, hid), lambda i: (0, 0)),
            pl.BlockSpec((1, hid), lambda i: (0, 0)),
            pl.BlockSpec((hid, ed), lambda i: (0, 0)),
            pl.BlockSpec((1, ed), lambda i: (0, 0)),
            pl.BlockSpec((1, ed), lambda i: (0, 0)),
            pl.BlockSpec((1, ed), lambda i: (0, 0)),
        ],
        out_specs=[
            pl.BlockSpec((eb, ed), lambda i: (i, 0)),
            pl.BlockSpec((eb, ed), lambda i: (i, 0)),
        ],
        out_shape=[
            jax.ShapeDtypeStruct((e, ed), jnp.float32),
            jax.ShapeDtypeStruct((e, ed), jnp.float32),
        ],
    )(g, ea, W1c, b1, W2, b2, ge2, be2ln)


def _sc_scatter(e2, col, zeros_init, ones_src, n):
    e, ed = e2.shape
    epw = e // _NW
    ch = 80
    nch = epw // ch
    rpt = n // _NS

    mesh = plsc.VectorSubcoreMesh(core_axis_name="c", subcore_axis_name="s")

    @functools.partial(
        pl.kernel,
        mesh=mesh,
        out_type=(
            jax.ShapeDtypeStruct((_NC, n, ed), jnp.float32),
            jax.ShapeDtypeStruct((_NC, n, ed), jnp.float32),
        ),
        scratch_types=[
            pltpu.VMEM((ch,), jnp.int32),
            pltpu.VMEM((ch, ed), jnp.float32),
            pltpu.VMEM((ch, ed), jnp.float32),
            pltpu.VMEM_SHARED((n, ed), jnp.float32),
            pltpu.VMEM_SHARED((n, ed), jnp.float32),
        ],
    )
    def k(e2_hbm, col_hbm, z_hbm, ones_hbm, seg_out, cnt_out,
          coli, ebuf, ones_v, seg_sh, cnt_sh):
        c = lax.axis_index("c")
        s = lax.axis_index("s")
        wid = s * _NC + c
        pltpu.sync_copy(z_hbm.at[pl.ds(s * rpt, rpt)], seg_sh.at[pl.ds(s * rpt, rpt)])
        pltpu.sync_copy(z_hbm.at[pl.ds(s * rpt, rpt)], cnt_sh.at[pl.ds(s * rpt, rpt)])
        pltpu.sync_copy(ones_hbm, ones_v)
        plsc.subcore_barrier()

        def body(i, carry):
            base = wid * epw + i * ch
            pltpu.sync_copy(col_hbm.at[pl.ds(base, ch)], coli)
            pltpu.sync_copy(e2_hbm.at[pl.ds(base, ch)], ebuf)
            pltpu.sync_copy(ebuf, seg_sh.at[coli], add=True)
            pltpu.sync_copy(ones_v, cnt_sh.at[coli], add=True)
            return carry

        lax.fori_loop(0, nch, body, 0)
        plsc.subcore_barrier()
        pltpu.sync_copy(seg_sh.at[pl.ds(s * rpt, rpt)],
                        seg_out.at[c, pl.ds(s * rpt, rpt)])
        pltpu.sync_copy(cnt_sh.at[pl.ds(s * rpt, rpt)],
                        cnt_out.at[c, pl.ds(s * rpt, rpt)])

    return k(e2, col, zeros_init, ones_src)


def _node_mlp(x, segp, cntp, W1a, W1b, b1, W2, b2, gxa, bxa):
    n, nd = x.shape
    ed = segp.shape[2]
    blk = 2000

    def body(x_ref, seg_ref, cnt_ref, w1a_ref, w1b_ref, b1_ref, w2_ref, b2_ref,
             gx_ref, bx_ref, out_ref):
        xv = x_ref[...]
        seg = seg_ref[0] + seg_ref[1]
        cnt = cnt_ref[0, :, 0:1] + cnt_ref[1, :, 0:1]
        agg = seg / jnp.maximum(cnt, 1.0)
        h1 = (jnp.dot(xv, w1a_ref[...], preferred_element_type=jnp.float32)
              + jnp.dot(agg, w1b_ref[...], preferred_element_type=jnp.float32)
              + b1_ref[...])
        hn = jnp.dot(jnp.maximum(h1, 0.0), w2_ref[...],
                     preferred_element_type=jnp.float32) + b2_ref[...]
        x2 = xv + hn
        m = jnp.mean(x2, axis=-1, keepdims=True)
        v = jnp.mean((x2 - m) ** 2, axis=-1, keepdims=True)
        out_ref[...] = (x2 - m) * lax.rsqrt(v + 1e-5) * gx_ref[...] + bx_ref[...]

    hid = W1a.shape[1]
    return pl.pallas_call(
        body,
        grid=(n // blk,),
        in_specs=[
            pl.BlockSpec((blk, nd), lambda i: (i, 0)),
            pl.BlockSpec((_NC, blk, ed), lambda i: (0, i, 0)),
            pl.BlockSpec((_NC, blk, ed), lambda i: (0, i, 0)),
            pl.BlockSpec((nd, hid), lambda i: (0, 0)),
            pl.BlockSpec((ed, hid), lambda i: (0, 0)),
            pl.BlockSpec((1, hid), lambda i: (0, 0)),
            pl.BlockSpec((hid, nd), lambda i: (0, 0)),
            pl.BlockSpec((1, nd), lambda i: (0, 0)),
            pl.BlockSpec((1, nd), lambda i: (0, 0)),
        ],
        out_specs=pl.BlockSpec((blk, nd), lambda i: (i, 0)),
        out_shape=jax.ShapeDtypeStruct((n, nd), jnp.float32),
    )(x, segp, cntp, W1a, W1b, b1, W2, b2, gxa, bxa)


def kernel(x, edge_index, edge_attr, batch, We1, be1, We2, be2,
           Wn1, bn1, Wn2, bn2, gx, bx, ge, bee):
    n, nd = x.shape
    e, ed = edge_attr.shape
    row = edge_index[0]
    col = edge_index[1]

    Wab = jnp.concatenate([We1[:nd], We1[nd:2 * nd]], axis=1)
    W1c = We1[2 * nd:]

    xs, xd = _node_proj(x, Wab)
    g = _sc_gather(xs, xd, row, col)
    e2, ln_e2 = _edge_mlp(
        g, edge_attr, W1c, be1.reshape(1, -1), We2, be2.reshape(1, -1),
        ge.reshape(1, -1), bee.reshape(1, -1))

    zeros_init = jnp.zeros((n, ed), jnp.float32)
    ones_src = jnp.ones((80, ed), jnp.float32)
    segp, cntp = _sc_scatter(e2, col, zeros_init, ones_src, n)

    ln_x2 = _node_mlp(
        x, segp, cntp, Wn1[:nd], Wn1[nd:], bn1.reshape(1, -1),
        Wn2, bn2.reshape(1, -1), gx.reshape(1, -1), bx.reshape(1, -1))
    return (ln_x2, ln_e2)


# speedup vs baseline: 4.7912x; 4.7912x over previous
import functools

import jax
import jax.numpy as jnp
from jax import lax
from jax.experimental import pallas as pl
from jax.experimental.pallas import tpu as pltpu
from jax.experimental.pallas import tpu_sc as plsc

_NC = 2
_NS = 16
_NW = _NC * _NS

_CH = 128
_DEPTH = 3


def _node_proj(x, Wab):
    n, nd = x.shape
    h2 = Wab.shape[1]
    blk = 1000

    def body(x_ref, w_ref, xs_ref, xd_ref):
        xw = jnp.dot(x_ref[...], w_ref[...], preferred_element_type=jnp.float32)
        xs_ref[...] = xw[:, : h2 // 2]
        xd_ref[...] = xw[:, h2 // 2 :]

    return pl.pallas_call(
        body,
        grid=(n // blk,),
        in_specs=[
            pl.BlockSpec((blk, nd), lambda i: (i, 0)),
            pl.BlockSpec((nd, h2), lambda i: (0, 0)),
        ],
        out_specs=[
            pl.BlockSpec((blk, h2 // 2), lambda i: (i, 0)),
            pl.BlockSpec((blk, h2 // 2), lambda i: (i, 0)),
        ],
        out_shape=[
            jax.ShapeDtypeStruct((n, h2 // 2), jnp.float32),
            jax.ShapeDtypeStruct((n, h2 // 2), jnp.float32),
        ],
    )(x, Wab)


def _sc_gather(xs, xd, row, col, zeros_init, ones_src, n):
    e = row.shape[0]
    hid = xs.shape[1]
    ned = zeros_init.shape[1]
    epw = e // _NW
    nfull = epw // _CH
    tail = epw - nfull * _CH
    ngrp = nfull // _DEPTH
    assert nfull % _DEPTH == 0 and ngrp >= 2
    rpt = n // _NS

    mesh = plsc.VectorSubcoreMesh(core_axis_name="c", subcore_axis_name="s")

    @functools.partial(
        pl.kernel,
        mesh=mesh,
        out_type=(
            jax.ShapeDtypeStruct((e, hid), jnp.float32),
            jax.ShapeDtypeStruct((_NC, n, ned), jnp.float32),
        ),
        scratch_types=[
            [pltpu.VMEM((_CH,), jnp.int32) for _ in range(_DEPTH)],
            [pltpu.VMEM((_CH,), jnp.int32) for _ in range(_DEPTH)],
            [pltpu.VMEM((_CH, hid), jnp.float32) for _ in range(_DEPTH)],
            pltpu.VMEM((tail,), jnp.int32),
            pltpu.VMEM((tail,), jnp.int32),
            pltpu.VMEM((tail, hid), jnp.float32),
            pltpu.VMEM((_CH, ned), jnp.float32),
            pltpu.VMEM_SHARED((n, ned), jnp.float32),
            [pltpu.SemaphoreType.DMA for _ in range(_DEPTH)],
            [pltpu.SemaphoreType.DMA for _ in range(_DEPTH)],
            [pltpu.SemaphoreType.DMA for _ in range(_DEPTH)],
            [pltpu.SemaphoreType.DMA for _ in range(_DEPTH)],
        ],
        compiler_params=pltpu.CompilerParams(use_tc_tiling_on_sc=False),
    )
    def k(xs_hbm, xd_hbm, row_hbm, col_hbm, z_hbm, ones_hbm, g_hbm, cnt_out,
          rowi, coli, gbuf, rowt, colt, gtail, ones_v, cnt_sh,
          idx_sem, xs_sem, xd_sem, st_sem):
        c_ax = lax.axis_index("c")
        s_ax = lax.axis_index("s")
        wid = s_ax * _NC + c_ax
        wbase = wid * epw

        pltpu.sync_copy(z_hbm.at[pl.ds(s_ax * rpt, rpt)],
                        cnt_sh.at[pl.ds(s_ax * rpt, rpt)])
        pltpu.sync_copy(ones_hbm, ones_v)
        plsc.subcore_barrier()

        def issue_idx(chunk, c):
            base = wbase + chunk * _CH
            pltpu.async_copy(row_hbm.at[pl.ds(base, _CH)], rowi[c], idx_sem[c])
            pltpu.async_copy(col_hbm.at[pl.ds(base, _CH)], coli[c], idx_sem[c])

        def wait_idx(chunk, c):
            base = wbase + chunk * _CH
            pltpu.make_async_copy(row_hbm.at[pl.ds(base, _CH)], rowi[c],
                                  idx_sem[c]).wait()
            pltpu.make_async_copy(col_hbm.at[pl.ds(base, _CH)], coli[c],
                                  idx_sem[c]).wait()

        for c in range(_DEPTH):
            issue_idx(c, c)
        for c in range(_DEPTH):
            wait_idx(c, c)
            pltpu.async_copy(xs_hbm.at[rowi[c]], gbuf[c], xs_sem[c])

        def group(i2, prep):
            for c in range(_DEPTH):
                pltpu.make_async_copy(xs_hbm.at[rowi[c]], gbuf[c],
                                      xs_sem[c]).wait()
                pltpu.async_copy(xd_hbm.at[coli[c]], gbuf[c], xd_sem[c],
                                 add=True)
            for c in range(_DEPTH):
                chunk = _DEPTH * i2 + c
                base = wbase + chunk * _CH
                pltpu.make_async_copy(xd_hbm.at[coli[c]], gbuf[c],
                                      xd_sem[c]).wait()
                pltpu.async_copy(gbuf[c], g_hbm.at[pl.ds(base, _CH)],
                                 st_sem[c])
                pltpu.sync_copy(ones_v, cnt_sh.at[coli[c]], add=True)
            if prep:
                for c in range(_DEPTH):
                    chunk = _DEPTH * i2 + c
                    base = wbase + chunk * _CH
                    issue_idx(chunk + _DEPTH, c)
                    pltpu.make_async_copy(gbuf[c], g_hbm.at[pl.ds(base, _CH)],
                                          st_sem[c]).wait()
                for c in range(_DEPTH):
                    chunk = _DEPTH * i2 + c
                    wait_idx(chunk + _DEPTH, c)
                    pltpu.async_copy(xs_hbm.at[rowi[c]], gbuf[c], xs_sem[c])

        def body(i2, carry):
            group(i2, prep=True)
            return carry

        lax.fori_loop(0, ngrp - 1, body, 0)
        group(ngrp - 1, prep=False)

        if tail:
            tbase = wbase + nfull * _CH
            pltpu.sync_copy(row_hbm.at[pl.ds(tbase, tail)], rowt)
            pltpu.sync_copy(col_hbm.at[pl.ds(tbase, tail)], colt)
            pltpu.sync_copy(xs_hbm.at[rowt], gtail)
            pltpu.sync_copy(xd_hbm.at[colt], gtail, add=True)
            pltpu.sync_copy(gtail, g_hbm.at[pl.ds(tbase, tail)])
            pltpu.sync_copy(ones_v.at[pl.ds(0, tail)], cnt_sh.at[colt],
                            add=True)

        for c in range(_DEPTH):
            chunk = _DEPTH * (ngrp - 1) + c
            base = wbase + chunk * _CH
            pltpu.make_async_copy(gbuf[c], g_hbm.at[pl.ds(base, _CH)],
                                  st_sem[c]).wait()

        plsc.subcore_barrier()
        pltpu.sync_copy(cnt_sh.at[pl.ds(s_ax * rpt, rpt)],
                        cnt_out.at[c_ax, pl.ds(s_ax * rpt, rpt)])

    return k(xs, xd, row, col, zeros_init, ones_src)


def _edge_mlp(g2, ea2, W1c2, b12, W22, b22, gmean, ge2, be2ln):
    e2rows, lanes = g2.shape
    eb = 1600

    def body(g_ref, ea_ref, w1_ref, b1_ref, w2_ref, b2_ref, gm_ref, ge_ref,
             be_ref, e2_ref, ln_ref):
        eav = ea_ref[...]
        h1 = g_ref[...] + jnp.dot(eav, w1_ref[...],
                                  preferred_element_type=jnp.float32) + b1_ref[...]
        h = jnp.dot(jnp.maximum(h1, 0.0), w2_ref[...],
                    preferred_element_type=jnp.float32) + b2_ref[...] + eav
        e2_ref[...] = h
        m = jnp.dot(h, gm_ref[...], preferred_element_type=jnp.float32)
        sq = jnp.dot(h * h, gm_ref[...], preferred_element_type=jnp.float32)
        v = sq - m * m
        ln_ref[...] = (h - m) * lax.rsqrt(v + 1e-5) * ge_ref[...] + be_ref[...]

    return pl.pallas_call(
        body,
        grid=(e2rows // eb,),
        in_specs=[
            pl.BlockSpec((eb, lanes), lambda i: (i, 0)),
            pl.BlockSpec((eb, 32), lambda i: (i, 0)),
            pl.BlockSpec((32, lanes), lambda i: (0, 0)),
            pl.BlockSpec((1, lanes), lambda i: (0, 0)),
            pl.BlockSpec((lanes, 32), lambda i: (0, 0)),
            pl.BlockSpec((1, 32), lambda i: (0, 0)),
            pl.BlockSpec((32, 32), lambda i: (0, 0)),
            pl.BlockSpec((1, 32), lambda i: (0, 0)),
            pl.BlockSpec((1, 32), lambda i: (0, 0)),
        ],
        out_specs=[
            pl.BlockSpec((eb, 32), lambda i: (i, 0)),
            pl.BlockSpec((eb, 32), lambda i: (i, 0)),
        ],
        out_shape=[
            jax.ShapeDtypeStruct((e2rows, 32), jnp.float32),
            jax.ShapeDtypeStruct((e2rows, 32), jnp.float32),
        ],
    )(g2, ea2, W1c2, b12, W22, b22, gmean, ge2, be2ln)


def _sc_scatter(e2, col, zeros_init, n):
    e, ed = e2.shape
    epw = e // _NW
    nfull = epw // _CH
    tail = epw - nfull * _CH
    ngrp = nfull // _DEPTH
    assert nfull % _DEPTH == 0 and ngrp >= 2
    rpt = n // _NS

    mesh = plsc.VectorSubcoreMesh(core_axis_name="c", subcore_axis_name="s")

    @functools.partial(
        pl.kernel,
        mesh=mesh,
        out_type=jax.ShapeDtypeStruct((_NC, n, ed), jnp.float32),
        scratch_types=[
            [pltpu.VMEM((_CH,), jnp.int32) for _ in range(_DEPTH)],
            [pltpu.VMEM((_CH, ed), jnp.float32) for _ in range(_DEPTH)],
            pltpu.VMEM((tail,), jnp.int32),
            pltpu.VMEM((tail, ed), jnp.float32),
            pltpu.VMEM_SHARED((n, ed), jnp.float32),
            [pltpu.SemaphoreType.DMA for _ in range(_DEPTH)],
        ],
        compiler_params=pltpu.CompilerParams(use_tc_tiling_on_sc=False),
    )
    def k(e2_hbm, col_hbm, z_hbm, seg_out,
          coli, ebuf, colt, etail, seg_sh, ld_sem):
        c_ax = lax.axis_index("c")
        s_ax = lax.axis_index("s")
        wid = s_ax * _NC + c_ax
        wbase = wid * epw

        pltpu.sync_copy(z_hbm.at[pl.ds(s_ax * rpt, rpt)],
                        seg_sh.at[pl.ds(s_ax * rpt, rpt)])
        plsc.subcore_barrier()

        def issue(chunk, c):
            base = wbase + chunk * _CH
            pltpu.async_copy(col_hbm.at[pl.ds(base, _CH)], coli[c], ld_sem[c])
            pltpu.async_copy(e2_hbm.at[pl.ds(base, _CH)], ebuf[c], ld_sem[c])

        def wait_ld(chunk, c):
            base = wbase + chunk * _CH
            pltpu.make_async_copy(col_hbm.at[pl.ds(base, _CH)], coli[c],
                                  ld_sem[c]).wait()
            pltpu.make_async_copy(e2_hbm.at[pl.ds(base, _CH)], ebuf[c],
                                  ld_sem[c]).wait()

        for c in range(_DEPTH):
            issue(c, c)

        def group(i2, prep):
            for c in range(_DEPTH):
                chunk = _DEPTH * i2 + c
                wait_ld(chunk, c)
                pltpu.sync_copy(ebuf[c], seg_sh.at[coli[c]], add=True)
                if prep:
                    issue(chunk + _DEPTH, c)

        def body(i2, carry):
            group(i2, prep=True)
            return carry

        lax.fori_loop(0, ngrp - 1, body, 0)
        group(ngrp - 1, prep=False)

        if tail:
            tbase = wbase + nfull * _CH
            pltpu.sync_copy(col_hbm.at[pl.ds(tbase, tail)], colt)
            pltpu.sync_copy(e2_hbm.at[pl.ds(tbase, tail)], etail)
            pltpu.sync_copy(etail, seg_sh.at[colt], add=True)

        plsc.subcore_barrier()
        pltpu.sync_copy(seg_sh.at[pl.ds(s_ax * rpt, rpt)],
                        seg_out.at[c_ax, pl.ds(s_ax * rpt, rpt)])

    return k(e2, col, zeros_init)


def _node_mlp(x, segp, cntp, W1a, W1b, b1, W2, b2, gxa, bxa):
    n, nd = x.shape
    ed = segp.shape[2]
    blk = 2000

    def body(x_ref, seg_ref, cnt_ref, w1a_ref, w1b_ref, b1_ref, w2_ref, b2_ref,
             gx_ref, bx_ref, out_ref):
        xv = x_ref[...]
        seg = seg_ref[0] + seg_ref[1]
        cnt = cnt_ref[0, :, 0:1] + cnt_ref[1, :, 0:1]
        agg = seg / jnp.maximum(cnt, 1.0)
        h1 = (jnp.dot(xv, w1a_ref[...], preferred_element_type=jnp.float32)
              + jnp.dot(agg, w1b_ref[...], preferred_element_type=jnp.float32)
              + b1_ref[...])
        hn = jnp.dot(jnp.maximum(h1, 0.0), w2_ref[...],
                     preferred_element_type=jnp.float32) + b2_ref[...]
        x2 = xv + hn
        m = jnp.mean(x2, axis=-1, keepdims=True)
        v = jnp.mean((x2 - m) ** 2, axis=-1, keepdims=True)
        out_ref[...] = (x2 - m) * lax.rsqrt(v + 1e-5) * gx_ref[...] + bx_ref[...]

    hid = W1a.shape[1]
    return pl.pallas_call(
        body,
        grid=(n // blk,),
        in_specs=[
            pl.BlockSpec((blk, nd), lambda i: (i, 0)),
            pl.BlockSpec((_NC, blk, ed), lambda i: (0, i, 0)),
            pl.BlockSpec((_NC, blk, ed), lambda i: (0, i, 0)),
            pl.BlockSpec((nd, hid), lambda i: (0, 0)),
            pl.BlockSpec((ed, hid), lambda i: (0, 0)),
            pl.BlockSpec((1, hid), lambda i: (0, 0)),
            pl.BlockSpec((hid, nd), lambda i: (0, 0)),
            pl.BlockSpec((1, nd), lambda i: (0, 0)),
            pl.BlockSpec((1, nd), lambda i: (0, 0)),
            pl.BlockSpec((1, nd), lambda i: (0, 0)),
        ],
        out_specs=pl.BlockSpec((blk, nd), lambda i: (i, 0)),
        out_shape=jax.ShapeDtypeStruct((n, nd), jnp.float32),
    )(x, segp, cntp, W1a, W1b, b1, W2, b2, gxa, bxa)


def kernel(x, edge_index, edge_attr, batch, We1, be1, We2, be2,
           Wn1, bn1, Wn2, bn2, gx, bx, ge, bee):
    n, nd = x.shape
    e, ed = edge_attr.shape
    row = edge_index[0]
    col = edge_index[1]

    Wab = jnp.concatenate([We1[:nd], We1[nd:2 * nd]], axis=1)
    W1c = We1[2 * nd:]
    hid = W1c.shape[1]

    zeros_init = jnp.zeros((n, ed), jnp.float32)
    ones_src = jnp.ones((_CH, ed), jnp.float32)

    xs, xd = _node_proj(x, Wab)
    g, cntp = _sc_gather(xs, xd, row, col, zeros_init, ones_src, n)

    eye2 = jnp.eye(2, dtype=jnp.float32)
    W1c2 = jnp.kron(eye2, W1c)
    W22 = jnp.kron(eye2, We2)
    gmean = jnp.kron(eye2, jnp.full((ed, ed), 1.0 / ed, jnp.float32))
    b12 = jnp.tile(be1, 2).reshape(1, -1)
    b22 = jnp.tile(be2, 2).reshape(1, -1)
    ge2 = jnp.tile(ge, 2).reshape(1, -1)
    be2ln = jnp.tile(bee, 2).reshape(1, -1)

    e2p, lnp = _edge_mlp(
        g.reshape(e // 2, 2 * hid), edge_attr.reshape(e // 2, 2 * ed),
        W1c2, b12, W22, b22, gmean, ge2, be2ln)
    e2 = e2p.reshape(e, ed)
    ln_e2 = lnp.reshape(e, ed)

    segp = _sc_scatter(e2, col, zeros_init, n)

    ln_x2 = _node_mlp(
        x, segp, cntp, Wn1[:nd], Wn1[nd:], bn1.reshape(1, -1),
        Wn2, bn2.reshape(1, -1), gx.reshape(1, -1), bx.reshape(1, -1))
    return (ln_x2, ln_e2)
